# baseline (speedup 1.0000x reference)
import jax
import jax.numpy as jnp
from jax import lax
from jax.experimental import pallas as pl
from jax.experimental.pallas import tpu as pltpu
from jax.experimental.pallas import tpu_sc as plsc

L = 16
NC, NS = 2, 16
NW = NC * NS
CHUNK = 2048


def _tap_table():
    taps = []
    for i in range(4):
        for j in range(4):
            lst = []
            for u in range(3):
                for v in range(3):
                    ii, jj = i + u - 1, j + v - 1
                    if 0 <= ii < 4 and 0 <= jj < 4:
                        lst.append((ii * 4 + jj, u * 3 + v))
            taps.append(lst)
    return taps


_TAPS = _tap_table()


def _sc_body(x_hbm, w_hbm, out_hbm, xin, xout, wv):
    c = lax.axis_index("c")
    s = lax.axis_index("s")
    wid = s * NC + c
    tiles_per_worker = x_hbm.shape[0] // (NW * L)
    n_chunks = tiles_per_worker // CHUNK

    pltpu.sync_copy(w_hbm, wv)
    w16 = wv[...]

    def bcast_lane(vec, k):
        return lax.gather(
            vec,
            jnp.full((L, 1), k, jnp.int32),
            lax.GatherDimensionNumbers(
                offset_dims=(), collapsed_slice_dims=(0,), start_index_map=(0,)
            ),
            slice_sizes=(1,),
            mode=lax.GatherScatterMode.PROMISE_IN_BOUNDS,
        )

    wvecs = [bcast_lane(w16, k) for k in range(9)]
    iota16 = lax.iota(jnp.int32, L) * L

    def chunk_body(ci, carry):
        base = (wid * tiles_per_worker + ci * CHUNK) * L
        pltpu.sync_copy(x_hbm.at[pl.ds(base, CHUNK * L)], xin)

        def group_body(g, carry2):
            gbase = g * (L * L)
            idx = [gbase + r + iota16 for r in range(L)]
            xs = [plsc.load_gather(xin, [idx[r]]) for r in range(L)]
            for r in range(L):
                acc = None
                for (rs, widx) in _TAPS[r]:
                    term = wvecs[widx] * xs[rs]
                    acc = term if acc is None else acc + term
                acc = jnp.where(xs[r] != 0.0, acc, 0.0)
                plsc.store_scatter(xout, [idx[r]], acc)
            return carry2

        lax.fori_loop(0, CHUNK // L, group_body, 0)
        pltpu.sync_copy(xout, out_hbm.at[pl.ds(base, CHUNK * L)])
        return carry

    lax.fori_loop(0, n_chunks, chunk_body, 0)


def kernel(x, W):
    n = x.shape[0]
    xf = x.reshape(n * 16)
    wf = jnp.concatenate([W.reshape(-1), jnp.zeros((7,), jnp.float32)])
    mesh = plsc.VectorSubcoreMesh(core_axis_name="c", subcore_axis_name="s")
    out = pl.kernel(
        _sc_body,
        out_type=jax.ShapeDtypeStruct((n * 16,), jnp.float32),
        mesh=mesh,
        compiler_params=pltpu.CompilerParams(needs_layout_passes=False),
        scratch_types=[
            pltpu.VMEM((CHUNK * L,), jnp.float32),
            pltpu.VMEM((CHUNK * L,), jnp.float32),
            pltpu.VMEM((L,), jnp.float32),
        ],
    )(xf, wf)
    return out.reshape(n, 4, 4, 1)


# speedup vs baseline: 41.3878x; 41.3878x over previous
import jax
import jax.numpy as jnp
from jax import lax
from jax.experimental import pallas as pl
from jax.experimental.pallas import tpu as pltpu
from jax.experimental.pallas import tpu_sc as plsc

L = 16
NC, NS = 2, 16
NW = NC * NS
CHUNK = 2048


def _tap_table():
    taps = []
    for i in range(4):
        for j in range(4):
            lst = []
            for u in range(3):
                for v in range(3):
                    ii, jj = i + u - 1, j + v - 1
                    if 0 <= ii < 4 and 0 <= jj < 4:
                        lst.append((ii * 4 + jj, u * 3 + v))
            taps.append(lst)
    return taps


_TAPS = _tap_table()


def _sc_body(x_hbm, w_hbm, out_hbm, xin, xout, wv, sem):
    c = lax.axis_index("c")
    s = lax.axis_index("s")
    wid = s * NC + c
    n = x_hbm.shape[0] // L
    tiles_per_worker = n // NW
    n_chunks = tiles_per_worker // CHUNK

    pltpu.sync_copy(w_hbm, wv)
    w16 = wv[...]

    def bcast_lane(vec, k):
        return lax.gather(
            vec,
            jnp.full((L, 1), k, jnp.int32),
            lax.GatherDimensionNumbers(
                offset_dims=(), collapsed_slice_dims=(0,), start_index_map=(0,)
            ),
            slice_sizes=(1,),
            mode=lax.GatherScatterMode.PROMISE_IN_BOUNDS,
        )

    wvecs = [bcast_lane(w16, k) for k in range(9)]

    def chunk_body(ci, carry):
        base = wid * tiles_per_worker + ci * CHUNK
        copies = [
            pltpu.async_copy(
                x_hbm.at[pl.ds(r * n + base, CHUNK)],
                xin.at[pl.ds(r * CHUNK, CHUNK)],
                sem,
            )
            for r in range(L)
        ]
        for cp in copies:
            cp.wait()

        def group_body(g, carry2):
            off = g * L
            xs = [xin[pl.ds(r * CHUNK + off, L)] for r in range(L)]
            for r in range(L):
                acc = None
                for (rs, widx) in _TAPS[r]:
                    term = wvecs[widx] * xs[rs]
                    acc = term if acc is None else acc + term
                acc = jnp.where(xs[r] != 0.0, acc, 0.0)
                xout[pl.ds(r * CHUNK + off, L)] = acc
            return carry2

        lax.fori_loop(0, CHUNK // L, group_body, 0)
        copies = [
            pltpu.async_copy(
                xout.at[pl.ds(r * CHUNK, CHUNK)],
                out_hbm.at[pl.ds(r * n + base, CHUNK)],
                sem,
            )
            for r in range(L)
        ]
        for cp in copies:
            cp.wait()
        return carry

    lax.fori_loop(0, n_chunks, chunk_body, 0)


def kernel(x, W):
    n = x.shape[0]
    xt = x.transpose(1, 2, 3, 0).reshape(16 * n)
    wf = jnp.concatenate([W.reshape(-1), jnp.zeros((7,), jnp.float32)])
    mesh = plsc.VectorSubcoreMesh(core_axis_name="c", subcore_axis_name="s")
    out = pl.kernel(
        _sc_body,
        out_type=jax.ShapeDtypeStruct((16 * n,), jnp.float32),
        mesh=mesh,
        compiler_params=pltpu.CompilerParams(needs_layout_passes=False),
        scratch_types=[
            pltpu.VMEM((16 * CHUNK,), jnp.float32),
            pltpu.VMEM((16 * CHUNK,), jnp.float32),
            pltpu.VMEM((L,), jnp.float32),
            pltpu.SemaphoreType.DMA,
        ],
    )(xt, wf)
    return out.reshape(4, 4, 1, n).transpose(3, 0, 1, 2)
